# baseline (speedup 1.0000x reference)
import functools

import jax
import jax.numpy as jnp
import numpy as np
from jax import lax
from jax.experimental import pallas as pl
from jax.experimental.pallas import tpu as pltpu
from jax.experimental.pallas import tpu_sc as plsc

B = 16384
D = 32
K = 20
NC = 2
NS = 16
L = 16
NW = NC * NS
BPW = B // NW
CB = 16
CROWS = CB * K
NCHUNK = BPW // CB
NPG = BPW // L


def _sc_body(in_emb, out_emb, cen_idx, ctx_idx, neg_idx,
             pos_out, neg_out,
             cen_idx_v, ctx_idx_v, neg_idx_v,
             cen_rows, ctx_rows, neg_buf, pos_v, neg_v, sem):
    wid = lax.axis_index("s") * NC + lax.axis_index("c")
    base = wid * BPW

    pltpu.sync_copy(cen_idx.at[pl.ds(base, BPW)], cen_idx_v)
    pltpu.sync_copy(ctx_idx.at[pl.ds(base, BPW)], ctx_idx_v)
    pltpu.sync_copy(neg_idx.at[pl.ds(base * K, BPW * K)], neg_idx_v)

    pltpu.async_copy(in_emb.at[cen_idx_v], cen_rows, sem).wait()
    pltpu.async_copy(out_emb.at[ctx_idx_v], ctx_rows, sem).wait()

    iota = lax.broadcasted_iota(jnp.int32, (L,), 0)

    @pl.loop(0, NPG)
    def _(g):
        bl = g * L + iota
        acc = jnp.zeros((L,), jnp.float32)
        for d in range(D):
            dv = jnp.full((L,), d, jnp.int32)
            cv = plsc.load_gather(cen_rows, [bl, dv])
            xv = plsc.load_gather(ctx_rows, [bl, dv])
            acc = acc + cv * xv
        pos_v[pl.ds(g * L, L)] = acc

    @pl.loop(0, NCHUNK)
    def _(c):
        pltpu.async_copy(
            out_emb.at[neg_idx_v.at[pl.ds(c * CROWS, CROWS)]], neg_buf, sem
        ).wait()
        bl = c * CB + iota
        rowb = iota * K
        accs = [jnp.zeros((L,), jnp.float32) for _ in range(K)]
        for d in range(D):
            dv = jnp.full((L,), d, jnp.int32)
            cv = plsc.load_gather(cen_rows, [bl, dv])
            for k in range(K):
                nv = plsc.load_gather(neg_buf, [rowb + k, dv])
                accs[k] = accs[k] + cv * nv
        for k in range(K):
            neg_v[pl.ds(k * BPW + c * CB, L)] = accs[k]

    pltpu.sync_copy(pos_v, pos_out.at[pl.ds(base, BPW)])
    pltpu.sync_copy(neg_v, neg_out.at[pl.ds(base * K, BPW * K)])


_sc_scores = functools.partial(
    pl.kernel,
    out_type=(
        jax.ShapeDtypeStruct((B,), jnp.float32),
        jax.ShapeDtypeStruct((B * K,), jnp.float32),
    ),
    mesh=plsc.VectorSubcoreMesh(core_axis_name="c", subcore_axis_name="s"),
    compiler_params=pltpu.CompilerParams(
        needs_layout_passes=False, use_tc_tiling_on_sc=False
    ),
    scratch_types=[
        pltpu.VMEM((BPW,), jnp.int32),
        pltpu.VMEM((BPW,), jnp.int32),
        pltpu.VMEM((BPW * K,), jnp.int32),
        pltpu.VMEM((BPW, D), jnp.float32),
        pltpu.VMEM((BPW, D), jnp.float32),
        pltpu.VMEM((CROWS, D), jnp.float32),
        pltpu.VMEM((BPW,), jnp.float32),
        pltpu.VMEM((BPW * K,), jnp.float32),
        pltpu.SemaphoreType.DMA,
    ],
)(_sc_body)


def _loss_body(pos_ref, neg_ref, out_ref):
    p = pos_ref[...]
    n = neg_ref[...]
    lp = jnp.log(jax.nn.sigmoid(p) + 1e-9)
    ln = jnp.log(jax.nn.sigmoid(-n) + 1e-9)
    out_ref[...] = (-(jnp.sum(lp) + jnp.sum(ln)) / np.float32(B)).reshape(1, 1)


_tc_loss = pl.pallas_call(
    _loss_body,
    out_shape=jax.ShapeDtypeStruct((1, 1), jnp.float32),
)


def kernel(input_embeddings, output_embeddings, center_words, context_words,
           negative_words):
    cen = center_words.astype(jnp.int32)
    ctx = context_words.astype(jnp.int32)
    neg = negative_words.astype(jnp.int32).reshape(-1)
    pos_s, neg_s = _sc_scores(input_embeddings, output_embeddings, cen, ctx, neg)
    loss = _tc_loss(pos_s.reshape(B // 128, 128), neg_s.reshape(B * K // 128, 128))
    return loss[0, 0]
